# Initial kernel scaffold; baseline (speedup 1.0000x reference)
import jax
import jax.numpy as jnp
from jax.experimental import pallas as pl


def kernel(x, edge_index, W1, a_src1, a_dst1, b1, W2, a_src2, a_dst2, b2):
    raise NotImplementedError("write your pallas kernel here")


# speedup vs baseline: 51.7364x; 51.7364x over previous
import functools

import jax
import jax.numpy as jnp
from jax import lax
from jax.experimental import pallas as pl
from jax.experimental.pallas import tpu as pltpu
from jax.experimental.pallas import tpu_sc as plsc

N = 10000
E = 320000
D_IN = 128
OUT = 16

NC = 2
NS = 16
NW = NC * NS
CHUNK = 128
NP = 10112
NCHUNK = -(-E // (NW * CHUNK))
EPAD = NW * CHUNK * NCHUNK


def _make_sc_edge_pass(wrow, nchunk):
    rps = NP // NS
    mesh = plsc.VectorSubcoreMesh(
        core_axis_name="c", subcore_axis_name="s", num_cores=NC, num_subcores=NS
    )
    nmsg = (wrow - 16) // 16

    @functools.partial(
        pl.kernel,
        out_type=jax.ShapeDtypeStruct((NC, NP, wrow), jnp.float32),
        mesh=mesh,
        scratch_types=[
            pltpu.VMEM((CHUNK,), jnp.int32),
            pltpu.VMEM((CHUNK,), jnp.int32),
            pltpu.VMEM((CHUNK, wrow), jnp.float32),
            pltpu.VMEM((CHUNK, 16), jnp.float32),
            pltpu.VMEM((CHUNK, wrow), jnp.float32),
            pltpu.VMEM_SHARED((NP, wrow), jnp.float32),
            pltpu.SemaphoreType.DMA,
        ],
        compiler_params=pltpu.CompilerParams(use_tc_tiling_on_sc=False),
    )
    def edge_pass(srctab, aldtab, sidx_h, didx_h, zeros_h, out_h,
                  sidx_v, didx_v, srows_v, arows_v, accrows_v, acc_sh, sem):
        c = lax.axis_index("c")
        s = lax.axis_index("s")
        w = c * NS + s
        base = pl.multiple_of(s * rps, 8)
        pltpu.sync_copy(zeros_h.at[pl.ds(base, rps)],
                        acc_sh.at[pl.ds(base, rps)])
        plsc.subcore_barrier()

        def chunk_body(j, carry):
            pltpu.sync_copy(sidx_h.at[w, j], sidx_v)
            pltpu.sync_copy(didx_h.at[w, j], didx_v)
            pltpu.async_copy(srctab.at[sidx_v], srows_v, sem).wait()
            pltpu.async_copy(aldtab.at[didx_v], arows_v, sem).wait()

            def edge_body(e, carry2):
                ald = arows_v[e, :]
                als = srows_v[e, pl.ds(wrow - 16, 16)]
                t = als + ald
                t = jnp.where(t >= 0.0, t, 0.2 * t)
                ee = jnp.exp(t)
                for k in range(nmsg):
                    accrows_v[e, pl.ds(k * 16, 16)] = (
                        srows_v[e, pl.ds(k * 16, 16)] * ee)
                accrows_v[e, pl.ds(wrow - 16, 16)] = ee
                return carry2

            lax.fori_loop(0, CHUNK, edge_body, 0)
            pltpu.sync_copy(accrows_v, acc_sh.at[didx_v], add=True)
            return carry

        lax.fori_loop(0, nchunk, chunk_body, 0)
        plsc.subcore_barrier()
        pltpu.sync_copy(acc_sh.at[pl.ds(base, rps)],
                        out_h.at[c, pl.ds(base, rps)])

    return edge_pass


def _mm_body(x_ref, w_ref, o1_ref, o2_ref, *, split):
    r = jnp.dot(x_ref[...], w_ref[...], preferred_element_type=jnp.float32)
    o1_ref[...] = r[:, :split]
    o2_ref[...] = r[:, split:]


def _project(x, wcat, split, grid=4):
    rows = NP // grid
    din = x.shape[1]
    cols = wcat.shape[1]
    return pl.pallas_call(
        functools.partial(_mm_body, split=split),
        grid=(grid,),
        in_specs=[
            pl.BlockSpec((rows, din), lambda i: (i, 0)),
            pl.BlockSpec((din, cols), lambda i: (0, 0)),
        ],
        out_specs=[
            pl.BlockSpec((rows, split), lambda i: (i, 0)),
            pl.BlockSpec((rows, cols - split), lambda i: (i, 0)),
        ],
        out_shape=[
            jax.ShapeDtypeStruct((NP, split), jnp.float32),
            jax.ShapeDtypeStruct((NP, cols - split), jnp.float32),
        ],
    )(x, wcat)


def _finalize1_body(p_ref, st_ref, ad_ref, w_ref, b_ref, o1_ref, o2_ref):
    als = st_ref[:, 64:80]
    ald = ad_ref[...]
    t = als + ald
    t = jnp.where(t >= 0.0, t, 0.2 * t)
    eself = jnp.exp(t)
    praw = p_ref[0] + p_ref[1]
    raw = praw[:, :64] + st_ref[:, :64] * jnp.tile(eself, (1, 4))
    denom = praw[:, 64:80] + eself
    z = raw / (jnp.tile(denom, (1, 4)) + 1e-16) + b_ref[...]
    z = jnp.where(z > 0.0, z, jnp.exp(jnp.minimum(z, 0.0)) - 1.0)
    r = jnp.dot(z, w_ref[...], preferred_element_type=jnp.float32)
    o1_ref[...] = r[:, :32]
    o2_ref[...] = r[:, 32:]


def _finalize1(p, srctab1, aldtab1, wcat2, b1p, grid=4):
    rows = NP // grid
    return pl.pallas_call(
        _finalize1_body,
        grid=(grid,),
        in_specs=[
            pl.BlockSpec((NC, rows, 80), lambda i: (0, i, 0)),
            pl.BlockSpec((rows, 80), lambda i: (i, 0)),
            pl.BlockSpec((rows, 16), lambda i: (i, 0)),
            pl.BlockSpec((64, 48), lambda i: (0, 0)),
            pl.BlockSpec((1, 64), lambda i: (0, 0)),
        ],
        out_specs=[
            pl.BlockSpec((rows, 32), lambda i: (i, 0)),
            pl.BlockSpec((rows, 16), lambda i: (i, 0)),
        ],
        out_shape=[
            jax.ShapeDtypeStruct((NP, 32), jnp.float32),
            jax.ShapeDtypeStruct((NP, 16), jnp.float32),
        ],
    )(p, srctab1, aldtab1, wcat2, b1p)


def _finalize2_body(p_ref, st_ref, ad_ref, b_ref, o_ref):
    als = st_ref[:, 16:32]
    t = als + ad_ref[...]
    t = jnp.where(t >= 0.0, t, 0.2 * t)
    eself = jnp.exp(t)
    praw = p_ref[0] + p_ref[1]
    raw = praw[:, :16] + st_ref[:, :16] * eself
    denom = praw[:, 16:32] + eself
    z = raw / (denom + 1e-16) + b_ref[...]
    m = jnp.max(z, axis=-1, keepdims=True)
    lse = jnp.log(jnp.sum(jnp.exp(z - m), axis=-1, keepdims=True))
    o_ref[...] = z - m - lse


def _finalize2(p2, srctab2, aldtab2, b2, grid=4):
    rows = NP // grid
    return pl.pallas_call(
        _finalize2_body,
        grid=(grid,),
        in_specs=[
            pl.BlockSpec((NC, rows, 32), lambda i: (0, i, 0)),
            pl.BlockSpec((rows, 32), lambda i: (i, 0)),
            pl.BlockSpec((rows, 16), lambda i: (i, 0)),
            pl.BlockSpec((1, 16), lambda i: (0, 0)),
        ],
        out_specs=pl.BlockSpec((rows, 16), lambda i: (i, 0)),
        out_shape=jax.ShapeDtypeStruct((NP, 16), jnp.float32),
    )(p2, srctab2, aldtab2, b2)


def kernel(x, edge_index, W1, a_src1, a_dst1, b1, W2, a_src2, a_dst2, b2):
    f32 = jnp.float32
    perm = jnp.arange(64).reshape(8, 8).T.reshape(-1)
    W1p = W1[:, perm]
    A1s = jnp.einsum("chf,hf->ch", W1.reshape(D_IN, 8, 8), a_src1[0])
    A1d = jnp.einsum("chf,hf->ch", W1.reshape(D_IN, 8, 8), a_dst1[0])
    wcat1 = jnp.concatenate([W1p, A1s, A1s, A1d, A1d], axis=1)
    b1p = b1[perm].reshape(1, 64)

    W2p = W2[perm, :]
    v2s = (W2 @ a_src2[0, 0])[perm].reshape(64, 1)
    v2d = (W2 @ a_dst2[0, 0])[perm].reshape(64, 1)
    wcat2 = jnp.concatenate(
        [W2p, jnp.tile(v2s, (1, 16)), jnp.tile(v2d, (1, 16))], axis=1)
    b2r = b2.reshape(1, 16)

    xp = jnp.pad(x, ((0, NP - N), (0, 0)))
    src = edge_index[0]
    dst = edge_index[1]
    padlen = EPAD - E
    srcp = jnp.concatenate(
        [src, jnp.full((padlen,), N, jnp.int32)]).reshape(NW, NCHUNK, CHUNK)
    dstp = jnp.concatenate(
        [dst, jnp.full((padlen,), N, jnp.int32)]).reshape(NW, NCHUNK, CHUNK)
    zeros80 = jnp.zeros((NP, 80), f32)
    zeros32 = jnp.zeros((NP, 32), f32)

    srctab1, aldtab1 = _project(xp, wcat1, 80)
    p1 = _make_sc_edge_pass(80, NCHUNK)(srctab1, aldtab1, srcp, dstp, zeros80)
    srctab2, aldtab2 = _finalize1(p1, srctab1, aldtab1, wcat2, b1p)

    p2 = _make_sc_edge_pass(32, NCHUNK)(srctab2, aldtab2, srcp, dstp, zeros32)
    out = _finalize2(p2, srctab2, aldtab2, b2r)
    return out[:N]


# speedup vs baseline: 98.8339x; 1.9103x over previous
import functools

import jax
import jax.numpy as jnp
from jax import lax
from jax.experimental import pallas as pl
from jax.experimental.pallas import tpu as pltpu
from jax.experimental.pallas import tpu_sc as plsc

N = 10000
E = 320000
D_IN = 128
OUT = 16

NC = 2
NS = 16
NW = NC * NS
CHUNK = 128
NP = 10112
NCHUNK = (-(-E // (NW * CHUNK)) + 1) // 2 * 2
EPAD = NW * CHUNK * NCHUNK


def _make_sc_edge_pass(wrow, nchunk):
    rps = NP // NS
    mesh = plsc.VectorSubcoreMesh(
        core_axis_name="c", subcore_axis_name="s", num_cores=NC, num_subcores=NS
    )
    nmsg = (wrow - 16) // 16

    @functools.partial(
        pl.kernel,
        out_type=jax.ShapeDtypeStruct((NC, NP, wrow), jnp.float32),
        mesh=mesh,
        scratch_types=[
            pltpu.VMEM((nchunk, CHUNK), jnp.int32),
            pltpu.VMEM((nchunk, CHUNK), jnp.int32),
            pltpu.VMEM((2, CHUNK, wrow), jnp.float32),
            pltpu.VMEM((2, CHUNK, 16), jnp.float32),
            pltpu.VMEM((2, CHUNK, wrow), jnp.float32),
            pltpu.VMEM_SHARED((NP, wrow), jnp.float32),
            pltpu.SemaphoreType.DMA((2,)),
            pltpu.SemaphoreType.DMA((2,)),
            pltpu.SemaphoreType.DMA((2,)),
        ],
        compiler_params=pltpu.CompilerParams(use_tc_tiling_on_sc=False),
    )
    def edge_pass(srctab, aldtab, sidx_h, didx_h, zeros_h, out_h,
                  sidx_vm, didx_vm, srows_v, arows_v, accrows_v, acc_sh,
                  gs_sem, ga_sem, sc_sem):
        c = lax.axis_index("c")
        s = lax.axis_index("s")
        w = c * NS + s
        base = pl.multiple_of(s * rps, 8)
        pltpu.sync_copy(zeros_h.at[pl.ds(base, rps)],
                        acc_sh.at[pl.ds(base, rps)])
        pltpu.sync_copy(sidx_h.at[w], sidx_vm)
        pltpu.sync_copy(didx_h.at[w], didx_vm)

        def issue_gather(j, b):
            pltpu.async_copy(srctab.at[sidx_vm.at[j]], srows_v.at[b],
                             gs_sem.at[b])
            pltpu.async_copy(aldtab.at[didx_vm.at[j]], arows_v.at[b],
                             ga_sem.at[b])

        issue_gather(0, 0)
        plsc.subcore_barrier()

        def pair_body(p, carry):
            for b in range(2):
                j = 2 * p + b
                nb = 1 - b

                @pl.when(j + 1 < nchunk)
                def _():
                    issue_gather(j + 1, nb)

                pltpu.make_async_copy(srctab.at[sidx_vm.at[j]],
                                      srows_v.at[b], gs_sem.at[b]).wait()
                pltpu.make_async_copy(aldtab.at[didx_vm.at[j]],
                                      arows_v.at[b], ga_sem.at[b]).wait()

                @pl.when(j >= 2)
                def _():
                    pltpu.make_async_copy(
                        accrows_v.at[b], acc_sh.at[didx_vm.at[j - 2]],
                        sc_sem.at[b]).wait()

                @plsc.parallel_loop(0, CHUNK, unroll=8)
                def _(e):
                    ald = arows_v[b, e, :]
                    als = srows_v[b, e, pl.ds(wrow - 16, 16)]
                    t = als + ald
                    t = jnp.where(t >= 0.0, t, 0.2 * t)
                    ee = jnp.exp(t)
                    for k in range(nmsg):
                        accrows_v[b, e, pl.ds(k * 16, 16)] = (
                            srows_v[b, e, pl.ds(k * 16, 16)] * ee)
                    accrows_v[b, e, pl.ds(wrow - 16, 16)] = ee

                pltpu.async_copy(accrows_v.at[b],
                                 acc_sh.at[didx_vm.at[j]],
                                 sc_sem.at[b], add=True)
            return carry

        lax.fori_loop(0, nchunk // 2, pair_body, 0)
        for b in range(2):
            pltpu.make_async_copy(accrows_v.at[b],
                                  acc_sh.at[didx_vm.at[nchunk - 2 + b]],
                                  sc_sem.at[b]).wait()
        plsc.subcore_barrier()
        pltpu.sync_copy(acc_sh.at[pl.ds(base, rps)],
                        out_h.at[c, pl.ds(base, rps)])

    return edge_pass


def _mm_body(x_ref, w_ref, o1_ref, o2_ref, *, split):
    r = jnp.dot(x_ref[...], w_ref[...], preferred_element_type=jnp.float32)
    o1_ref[...] = r[:, :split]
    o2_ref[...] = r[:, split:]


def _project(x, wcat, split, grid=4):
    rows = NP // grid
    din = x.shape[1]
    cols = wcat.shape[1]
    return pl.pallas_call(
        functools.partial(_mm_body, split=split),
        grid=(grid,),
        in_specs=[
            pl.BlockSpec((rows, din), lambda i: (i, 0)),
            pl.BlockSpec((din, cols), lambda i: (0, 0)),
        ],
        out_specs=[
            pl.BlockSpec((rows, split), lambda i: (i, 0)),
            pl.BlockSpec((rows, cols - split), lambda i: (i, 0)),
        ],
        out_shape=[
            jax.ShapeDtypeStruct((NP, split), jnp.float32),
            jax.ShapeDtypeStruct((NP, cols - split), jnp.float32),
        ],
    )(x, wcat)


def _finalize1_body(p_ref, st_ref, ad_ref, w_ref, b_ref, o1_ref, o2_ref):
    als = st_ref[:, 64:80]
    ald = ad_ref[...]
    t = als + ald
    t = jnp.where(t >= 0.0, t, 0.2 * t)
    eself = jnp.exp(t)
    praw = p_ref[0] + p_ref[1]
    raw = praw[:, :64] + st_ref[:, :64] * jnp.tile(eself, (1, 4))
    denom = praw[:, 64:80] + eself
    z = raw / (jnp.tile(denom, (1, 4)) + 1e-16) + b_ref[...]
    z = jnp.where(z > 0.0, z, jnp.exp(jnp.minimum(z, 0.0)) - 1.0)
    r = jnp.dot(z, w_ref[...], preferred_element_type=jnp.float32)
    o1_ref[...] = r[:, :32]
    o2_ref[...] = r[:, 32:]


def _finalize1(p, srctab1, aldtab1, wcat2, b1p, grid=4):
    rows = NP // grid
    return pl.pallas_call(
        _finalize1_body,
        grid=(grid,),
        in_specs=[
            pl.BlockSpec((NC, rows, 80), lambda i: (0, i, 0)),
            pl.BlockSpec((rows, 80), lambda i: (i, 0)),
            pl.BlockSpec((rows, 16), lambda i: (i, 0)),
            pl.BlockSpec((64, 48), lambda i: (0, 0)),
            pl.BlockSpec((1, 64), lambda i: (0, 0)),
        ],
        out_specs=[
            pl.BlockSpec((rows, 32), lambda i: (i, 0)),
            pl.BlockSpec((rows, 16), lambda i: (i, 0)),
        ],
        out_shape=[
            jax.ShapeDtypeStruct((NP, 32), jnp.float32),
            jax.ShapeDtypeStruct((NP, 16), jnp.float32),
        ],
    )(p, srctab1, aldtab1, wcat2, b1p)


def _finalize2_body(p_ref, st_ref, ad_ref, b_ref, o_ref):
    als = st_ref[:, 16:32]
    t = als + ad_ref[...]
    t = jnp.where(t >= 0.0, t, 0.2 * t)
    eself = jnp.exp(t)
    praw = p_ref[0] + p_ref[1]
    raw = praw[:, :16] + st_ref[:, :16] * eself
    denom = praw[:, 16:32] + eself
    z = raw / (denom + 1e-16) + b_ref[...]
    m = jnp.max(z, axis=-1, keepdims=True)
    lse = jnp.log(jnp.sum(jnp.exp(z - m), axis=-1, keepdims=True))
    o_ref[...] = z - m - lse


def _finalize2(p2, srctab2, aldtab2, b2, grid=4):
    rows = NP // grid
    return pl.pallas_call(
        _finalize2_body,
        grid=(grid,),
        in_specs=[
            pl.BlockSpec((NC, rows, 32), lambda i: (0, i, 0)),
            pl.BlockSpec((rows, 32), lambda i: (i, 0)),
            pl.BlockSpec((rows, 16), lambda i: (i, 0)),
            pl.BlockSpec((1, 16), lambda i: (0, 0)),
        ],
        out_specs=pl.BlockSpec((rows, 16), lambda i: (i, 0)),
        out_shape=jax.ShapeDtypeStruct((NP, 16), jnp.float32),
    )(p2, srctab2, aldtab2, b2)


def kernel(x, edge_index, W1, a_src1, a_dst1, b1, W2, a_src2, a_dst2, b2):
    f32 = jnp.float32
    perm = jnp.arange(64).reshape(8, 8).T.reshape(-1)
    W1p = W1[:, perm]
    A1s = jnp.einsum("chf,hf->ch", W1.reshape(D_IN, 8, 8), a_src1[0])
    A1d = jnp.einsum("chf,hf->ch", W1.reshape(D_IN, 8, 8), a_dst1[0])
    wcat1 = jnp.concatenate([W1p, A1s, A1s, A1d, A1d], axis=1)
    b1p = b1[perm].reshape(1, 64)

    W2p = W2[perm, :]
    v2s = (W2 @ a_src2[0, 0])[perm].reshape(64, 1)
    v2d = (W2 @ a_dst2[0, 0])[perm].reshape(64, 1)
    wcat2 = jnp.concatenate(
        [W2p, jnp.tile(v2s, (1, 16)), jnp.tile(v2d, (1, 16))], axis=1)
    b2r = b2.reshape(1, 16)

    xp = jnp.pad(x, ((0, NP - N), (0, 0)))
    src = edge_index[0]
    dst = edge_index[1]
    padlen = EPAD - E
    srcp = jnp.concatenate(
        [src, jnp.full((padlen,), N, jnp.int32)]).reshape(NW, NCHUNK, CHUNK)
    dstp = jnp.concatenate(
        [dst, jnp.full((padlen,), N, jnp.int32)]).reshape(NW, NCHUNK, CHUNK)
    zeros80 = jnp.zeros((NP, 80), f32)
    zeros32 = jnp.zeros((NP, 32), f32)

    srctab1, aldtab1 = _project(xp, wcat1, 80)
    p1 = _make_sc_edge_pass(80, NCHUNK)(srctab1, aldtab1, srcp, dstp, zeros80)
    srctab2, aldtab2 = _finalize1(p1, srctab1, aldtab1, wcat2, b1p)

    p2 = _make_sc_edge_pass(32, NCHUNK)(srctab2, aldtab2, srcp, dstp, zeros32)
    out = _finalize2(p2, srctab2, aldtab2, b2r)
    return out[:N]


# speedup vs baseline: 203.7934x; 2.0620x over previous
import functools

import jax
import jax.numpy as jnp
from jax import lax
from jax.experimental import pallas as pl
from jax.experimental.pallas import tpu as pltpu
from jax.experimental.pallas import tpu_sc as plsc

N = 10000
E = 320000
D_IN = 128
OUT = 16

NC = 2
NS = 16
NW = NC * NS
CHUNK = 128
NP = 10112
NCHUNK = (-(-E // (NW * CHUNK)) + 1) // 2 * 2
EPAD = NW * CHUNK * NCHUNK


def _make_sc_edge_pass(wrow, nchunk):
    rps = NP // NS
    mesh = plsc.VectorSubcoreMesh(
        core_axis_name="c", subcore_axis_name="s", num_cores=NC, num_subcores=NS
    )
    nmsg = (wrow - 16) // 16

    @functools.partial(
        pl.kernel,
        out_type=jax.ShapeDtypeStruct((NC, NP, wrow), jnp.float32),
        mesh=mesh,
        scratch_types=[
            pltpu.VMEM((nchunk, CHUNK), jnp.int32),
            pltpu.VMEM((nchunk, CHUNK), jnp.int32),
            pltpu.VMEM((2, CHUNK, wrow), jnp.float32),
            pltpu.VMEM((2, CHUNK, 16), jnp.float32),
            pltpu.VMEM((2, CHUNK, wrow), jnp.float32),
            pltpu.VMEM_SHARED((NP, wrow), jnp.float32),
            pltpu.SemaphoreType.DMA((2,)),
            pltpu.SemaphoreType.DMA((2,)),
            pltpu.SemaphoreType.DMA((2,)),
        ],
        compiler_params=pltpu.CompilerParams(use_tc_tiling_on_sc=False),
    )
    def edge_pass(srctab, aldtab, sidx_h, didx_h, zeros_h, out_h,
                  sidx_vm, didx_vm, srows_v, arows_v, accrows_v, acc_sh,
                  gs_sem, ga_sem, sc_sem):
        c = lax.axis_index("c")
        s = lax.axis_index("s")
        w = c * NS + s
        base = pl.multiple_of(s * rps, 8)
        pltpu.sync_copy(zeros_h.at[pl.ds(base, rps)],
                        acc_sh.at[pl.ds(base, rps)])
        pltpu.sync_copy(sidx_h.at[w], sidx_vm)
        pltpu.sync_copy(didx_h.at[w], didx_vm)

        def issue_gather(j, b):
            pltpu.async_copy(srctab.at[sidx_vm.at[j]], srows_v.at[b],
                             gs_sem.at[b])
            pltpu.async_copy(aldtab.at[didx_vm.at[j]], arows_v.at[b],
                             ga_sem.at[b])

        issue_gather(0, 0)
        plsc.subcore_barrier()

        def pair_body(p, carry):
            for b in range(2):
                j = 2 * p + b
                nb = 1 - b

                @pl.when(j + 1 < nchunk)
                def _():
                    issue_gather(j + 1, nb)

                pltpu.make_async_copy(srctab.at[sidx_vm.at[j]],
                                      srows_v.at[b], gs_sem.at[b]).wait()
                pltpu.make_async_copy(aldtab.at[didx_vm.at[j]],
                                      arows_v.at[b], ga_sem.at[b]).wait()

                @pl.when(j >= 2)
                def _():
                    pltpu.make_async_copy(
                        accrows_v.at[b], acc_sh.at[didx_vm.at[j - 2]],
                        sc_sem.at[b]).wait()

                @plsc.parallel_loop(0, CHUNK, unroll=8)
                def _(e):
                    ald = arows_v[b, e, :]
                    als = srows_v[b, e, pl.ds(wrow - 16, 16)]
                    t = als + ald
                    t = jnp.where(t >= 0.0, t, 0.2 * t)
                    ee = jnp.exp(t)
                    for k in range(nmsg):
                        accrows_v[b, e, pl.ds(k * 16, 16)] = (
                            srows_v[b, e, pl.ds(k * 16, 16)] * ee)
                    accrows_v[b, e, pl.ds(wrow - 16, 16)] = ee

                pltpu.async_copy(accrows_v.at[b],
                                 acc_sh.at[didx_vm.at[j]],
                                 sc_sem.at[b], add=True)
            return carry

        lax.fori_loop(0, nchunk // 2, pair_body, 0)
        for b in range(2):
            pltpu.make_async_copy(accrows_v.at[b],
                                  acc_sh.at[didx_vm.at[nchunk - 2 + b]],
                                  sc_sem.at[b]).wait()
        plsc.subcore_barrier()
        pltpu.sync_copy(acc_sh.at[pl.ds(base, rps)],
                        out_h.at[c, pl.ds(base, rps)])

    return edge_pass


def _mm_body(x_ref, w_ref, o1_ref, o2_ref, *, split):
    r = jnp.dot(x_ref[...], w_ref[...], preferred_element_type=jnp.float32)
    o1_ref[...] = r[:, :split]
    o2_ref[...] = r[:, split:]


def _project(x, wcat, split, grid=4):
    rows = NP // grid
    din = x.shape[1]
    cols = wcat.shape[1]
    return pl.pallas_call(
        functools.partial(_mm_body, split=split),
        grid=(grid,),
        in_specs=[
            pl.BlockSpec((rows, din), lambda i: (i, 0)),
            pl.BlockSpec((din, cols), lambda i: (0, 0)),
        ],
        out_specs=[
            pl.BlockSpec((rows, split), lambda i: (i, 0)),
            pl.BlockSpec((rows, cols - split), lambda i: (i, 0)),
        ],
        out_shape=[
            jax.ShapeDtypeStruct((NP, split), jnp.float32),
            jax.ShapeDtypeStruct((NP, cols - split), jnp.float32),
        ],
    )(x, wcat)


def _finalize1_body(p_ref, st_ref, ad_ref, w_ref, b_ref, o1_ref, o2_ref):
    als = st_ref[:, 64:80]
    ald = ad_ref[...]
    t = als + ald
    t = jnp.where(t >= 0.0, t, 0.2 * t)
    eself = jnp.exp(t)
    praw = p_ref[0] + p_ref[1]
    raw = praw[:, :64] + st_ref[:, :64] * jnp.tile(eself, (1, 4))
    denom = praw[:, 64:80] + eself
    z = raw / (jnp.tile(denom, (1, 4)) + 1e-16) + b_ref[...]
    z = jnp.where(z > 0.0, z, jnp.exp(jnp.minimum(z, 0.0)) - 1.0)
    r = jnp.dot(z, w_ref[...], preferred_element_type=jnp.float32)
    o1_ref[...] = r[:, :32]
    o2_ref[...] = r[:, 32:]


def _finalize1(p, srctab1, aldtab1, wcat2, b1p, grid=4):
    rows = NP // grid
    return pl.pallas_call(
        _finalize1_body,
        grid=(grid,),
        in_specs=[
            pl.BlockSpec((NC, rows, 80), lambda i: (0, i, 0)),
            pl.BlockSpec((rows, 80), lambda i: (i, 0)),
            pl.BlockSpec((rows, 16), lambda i: (i, 0)),
            pl.BlockSpec((64, 48), lambda i: (0, 0)),
            pl.BlockSpec((1, 64), lambda i: (0, 0)),
        ],
        out_specs=[
            pl.BlockSpec((rows, 32), lambda i: (i, 0)),
            pl.BlockSpec((rows, 16), lambda i: (i, 0)),
        ],
        out_shape=[
            jax.ShapeDtypeStruct((NP, 32), jnp.float32),
            jax.ShapeDtypeStruct((NP, 16), jnp.float32),
        ],
    )(p, srctab1, aldtab1, wcat2, b1p)


def _finalize2_body(p_ref, st_ref, ad_ref, b_ref, o_ref):
    als = st_ref[:, 16:32]
    t = als + ad_ref[...]
    t = jnp.where(t >= 0.0, t, 0.2 * t)
    eself = jnp.exp(t)
    praw = p_ref[0] + p_ref[1]
    raw = praw[:, :16] + st_ref[:, :16] * eself
    denom = praw[:, 16:32] + eself
    z = raw / (denom + 1e-16) + b_ref[...]
    m = jnp.max(z, axis=-1, keepdims=True)
    lse = jnp.log(jnp.sum(jnp.exp(z - m), axis=-1, keepdims=True))
    o_ref[...] = z - m - lse


def _finalize2(p2, srctab2, aldtab2, b2, grid=4):
    rows = NP // grid
    return pl.pallas_call(
        _finalize2_body,
        grid=(grid,),
        in_specs=[
            pl.BlockSpec((NC, rows, 32), lambda i: (0, i, 0)),
            pl.BlockSpec((rows, 32), lambda i: (i, 0)),
            pl.BlockSpec((rows, 16), lambda i: (i, 0)),
            pl.BlockSpec((1, 16), lambda i: (0, 0)),
        ],
        out_specs=pl.BlockSpec((rows, 16), lambda i: (i, 0)),
        out_shape=jax.ShapeDtypeStruct((NP, 16), jnp.float32),
    )(p2, srctab2, aldtab2, b2)


def kernel(x, edge_index, W1, a_src1, a_dst1, b1, W2, a_src2, a_dst2, b2):
    f32 = jnp.float32
    perm = jnp.arange(64).reshape(8, 8).T.reshape(-1)
    W1p = W1[:, perm]
    A1s = jnp.einsum("chf,hf->ch", W1.reshape(D_IN, 8, 8), a_src1[0])
    A1d = jnp.einsum("chf,hf->ch", W1.reshape(D_IN, 8, 8), a_dst1[0])
    wcat1 = jnp.concatenate([W1p, A1s, A1s, A1d, A1d], axis=1)
    b1p = b1[perm].reshape(1, 64)

    W2p = W2[perm, :]
    v2s = (W2 @ a_src2[0, 0])[perm].reshape(64, 1)
    v2d = (W2 @ a_dst2[0, 0])[perm].reshape(64, 1)
    wcat2 = jnp.concatenate(
        [W2p, jnp.tile(v2s, (1, 16)), jnp.tile(v2d, (1, 16))], axis=1)
    b2r = b2.reshape(1, 16)

    xp = jnp.pad(x, ((0, NP - N), (0, 0)))
    src = edge_index[0]
    dst = edge_index[1]
    padlen = EPAD - E
    trash = (N + jnp.arange(padlen, dtype=jnp.int32) % (NP - N)).astype(jnp.int32)
    srcp = jnp.concatenate([src, trash]).reshape(NW, NCHUNK, CHUNK)
    dstp = jnp.concatenate([dst, trash]).reshape(NW, NCHUNK, CHUNK)
    zeros80 = jnp.zeros((NP, 80), f32)
    zeros32 = jnp.zeros((NP, 32), f32)

    srctab1, aldtab1 = _project(xp, wcat1, 80)
    p1 = _make_sc_edge_pass(80, NCHUNK)(srctab1, aldtab1, srcp, dstp, zeros80)
    srctab2, aldtab2 = _finalize1(p1, srctab1, aldtab1, wcat2, b1p)

    p2 = _make_sc_edge_pass(32, NCHUNK)(srctab2, aldtab2, srcp, dstp, zeros32)
    out = _finalize2(p2, srctab2, aldtab2, b2r)
    return out[:N]
